# Initial kernel scaffold; baseline (speedup 1.0000x reference)
import functools

import jax
import jax.numpy as jnp
from jax import lax
from jax.experimental import pallas as pl
from jax.experimental.pallas import tpu as pltpu
from jax.experimental.pallas import tpu_sc as plsc

M = 65536
N = 16384
E = 8
SH = 16
HID = 128
IN_DIM = 3 + 3 + 4 + 1 + SH * 3 + 32 + 1
OUT_DIM = 3 + 3 + 4 + 1 + SH * 3
F_PAD = 128
O_PAD = 64

NC = 2
NS = 16
NW = NC * NS
B_W = N // NW
CH = 128
NCH = B_W // CH


def _sc_gather_body(idx_hbm, xyz, scl, rot, opa, shf, emb, seg,
                    o_xyz, o_scl, o_rot, o_opa, o_shf, o_emb, o_seg,
                    idx_v, b_xyz, b_scl, b_rot, b_opa, b_shf, b_emb, b_seg,
                    sem):
  wid = lax.axis_index("s") * NC + lax.axis_index("c")
  base = wid * B_W
  pltpu.sync_copy(idx_hbm.at[pl.ds(wid * NCH, NCH)], idx_v)
  tables = (xyz, scl, rot, opa, shf, emb, seg)
  bufs = (b_xyz, b_scl, b_rot, b_opa, b_shf, b_emb, b_seg)
  outs = (o_xyz, o_scl, o_rot, o_opa, o_shf, o_emb, o_seg)
  for j in range(NCH):
    cps = [
        pltpu.async_copy(t.at[idx_v.at[j]], b.at[pl.ds(j * CH, CH)], sem)
        for t, b in zip(tables, bufs)
    ]
    for c in cps:
      c.wait()
  for b, o in zip(bufs, outs):
    pltpu.sync_copy(b, o.at[pl.ds(base, B_W)])


@jax.jit
def _sc_gather(idx2d, xyz, scl, rot, opa, shf, emb, seg2d):
  mesh = plsc.VectorSubcoreMesh(core_axis_name="c", subcore_axis_name="s")
  out_type = [
      jax.ShapeDtypeStruct((N, 3), jnp.float32),
      jax.ShapeDtypeStruct((N, 3), jnp.float32),
      jax.ShapeDtypeStruct((N, 4), jnp.float32),
      jax.ShapeDtypeStruct((N, 1), jnp.float32),
      jax.ShapeDtypeStruct((N, 48), jnp.float32),
      jax.ShapeDtypeStruct((N, 32), jnp.float32),
      jax.ShapeDtypeStruct((N, 1), jnp.int32),
  ]
  scratch = [
      pltpu.VMEM((NCH, CH), jnp.int32),
      pltpu.VMEM((B_W, 3), jnp.float32),
      pltpu.VMEM((B_W, 3), jnp.float32),
      pltpu.VMEM((B_W, 4), jnp.float32),
      pltpu.VMEM((B_W, 1), jnp.float32),
      pltpu.VMEM((B_W, 48), jnp.float32),
      pltpu.VMEM((B_W, 32), jnp.float32),
      pltpu.VMEM((B_W, 1), jnp.int32),
      pltpu.SemaphoreType.DMA,
  ]
  fn = pl.kernel(
      _sc_gather_body, out_type=out_type, mesh=mesh, scratch_types=scratch)
  return fn(idx2d, xyz, scl, rot, opa, shf, emb, seg2d)


def _tc_mlp_body(feat_ref, seg_ref, w1_ref, b1_ref, w2_ref, b2_ref, out_ref):
  x = feat_ref[...]
  seg = seg_ref[...]
  acc = jnp.zeros(out_ref.shape, dtype=jnp.float32)
  for e in range(E):
    h = jnp.maximum(
        jnp.dot(x, w1_ref[e], preferred_element_type=jnp.float32)
        + b1_ref[e], 0.0)
    d = jnp.dot(h, w2_ref[e], preferred_element_type=jnp.float32) + b2_ref[e]
    acc = jnp.where(seg == e, d, acc)
  out_ref[...] = x[:, :O_PAD] + acc


@jax.jit
def _tc_mlp(feat, seg, w1p, b1p, w2p, b2p):
  bm = 512
  grid = (N // bm,)
  return pl.pallas_call(
      _tc_mlp_body,
      grid=grid,
      in_specs=[
          pl.BlockSpec((bm, F_PAD), lambda i: (i, 0)),
          pl.BlockSpec((bm, 1), lambda i: (i, 0)),
          pl.BlockSpec((E, F_PAD, HID), lambda i: (0, 0, 0)),
          pl.BlockSpec((E, 1, HID), lambda i: (0, 0, 0)),
          pl.BlockSpec((E, HID, O_PAD), lambda i: (0, 0, 0)),
          pl.BlockSpec((E, 1, O_PAD), lambda i: (0, 0, 0)),
      ],
      out_specs=pl.BlockSpec((bm, O_PAD), lambda i: (i, 0)),
      out_shape=jax.ShapeDtypeStruct((N, O_PAD), jnp.float32),
  )(feat, seg, w1p, b1p, w2p, b2p)


def kernel(idx_tensor, time_values, xyz, scaling, rotation, opacity, shs,
           embedding, seg_id_g, W1, b1, W2, b2):
  idx2d = idx_tensor.reshape(N // CH, CH)
  shf_t = shs.reshape(M, SH * 3)
  seg2d = seg_id_g.reshape(M, 1)
  g_xyz, g_scl, g_rot, g_opa, g_shf, g_emb, g_seg = _sc_gather(
      idx2d, xyz, scaling, rotation, opacity, shf_t, embedding, seg2d)

  t = time_values.reshape(N, 1)
  feat = jnp.concatenate(
      [g_xyz, g_scl, g_rot, g_opa, g_shf, g_emb, t,
       jnp.zeros((N, F_PAD - IN_DIM), jnp.float32)], axis=-1)

  w1p = jnp.concatenate(
      [W1, jnp.zeros((E, F_PAD - IN_DIM, HID), jnp.float32)], axis=1)
  b1p = b1.reshape(E, 1, HID)
  w2p = jnp.concatenate(
      [W2, jnp.zeros((E, HID, O_PAD - OUT_DIM), jnp.float32)], axis=2)
  b2p = jnp.concatenate(
      [b2, jnp.zeros((E, O_PAD - OUT_DIM), jnp.float32)],
      axis=1).reshape(E, 1, O_PAD)

  out = _tc_mlp(feat, g_seg, w1p, b1p, w2p, b2p)

  means_o = out[:, 0:3]
  scales_o = out[:, 3:6]
  rot_o = out[:, 6:10]
  opa_o = out[:, 10:11]
  shs_o = out[:, 11:OUT_DIM].reshape(N, SH, 3)
  return (means_o, scales_o, rot_o, opa_o, shs_o)


# speedup vs baseline: 1.0410x; 1.0410x over previous
import jax
import jax.numpy as jnp
from jax import lax
from jax.experimental import pallas as pl
from jax.experimental.pallas import tpu as pltpu
from jax.experimental.pallas import tpu_sc as plsc

M = 65536
N = 16384
E = 8
SH = 16
HID = 128
IN_DIM = 3 + 3 + 4 + 1 + SH * 3 + 32 + 1
OUT_DIM = 3 + 3 + 4 + 1 + SH * 3
F_PAD = 128
O_PAD = 64

NC = 2
NS = 16
NW = NC * NS
B_W = N // NW
CH = 128
NCH = B_W // CH


def _sc_gather_body(idx_hbm, t16, t48, t32,
                    o16, o48, o32,
                    idx_v, b16, b48, b32, sem):
  wid = lax.axis_index("s") * NC + lax.axis_index("c")
  base = wid * B_W
  pltpu.sync_copy(idx_hbm.at[pl.ds(wid * NCH, NCH)], idx_v)
  for j in range(NCH):
    cps = [
        pltpu.async_copy(t.at[idx_v.at[j]], b.at[pl.ds(j * CH, CH)], sem)
        for t, b in ((t16, b16), (t48, b48), (t32, b32))
    ]
    for c in cps:
      c.wait()
  for b, o in ((b16, o16), (b48, o48), (b32, o32)):
    pltpu.sync_copy(b, o.at[pl.ds(base, B_W)])


@jax.jit
def _sc_gather(idx2d, tab16, tab48, tab32):
  mesh = plsc.VectorSubcoreMesh(core_axis_name="c", subcore_axis_name="s")
  out_type = [
      jax.ShapeDtypeStruct((N, 16), jnp.float32),
      jax.ShapeDtypeStruct((N, 48), jnp.float32),
      jax.ShapeDtypeStruct((N, 32), jnp.float32),
  ]
  scratch = [
      pltpu.VMEM((NCH, CH), jnp.int32),
      pltpu.VMEM((B_W, 16), jnp.float32),
      pltpu.VMEM((B_W, 48), jnp.float32),
      pltpu.VMEM((B_W, 32), jnp.float32),
      pltpu.SemaphoreType.DMA,
  ]
  fn = pl.kernel(
      _sc_gather_body, out_type=out_type, mesh=mesh, scratch_types=scratch,
      compiler_params=pltpu.CompilerParams(use_tc_tiling_on_sc=False))
  return fn(idx2d, tab16, tab48, tab32)


def _tc_mlp_body(feat_ref, w1_ref, b1_ref, w2_ref, b2_ref, out_ref):
  x = feat_ref[...]
  seg = x[:, 11:12]
  acc = jnp.zeros(out_ref.shape, dtype=jnp.float32)
  for e in range(E):
    h = jnp.maximum(
        jnp.dot(x, w1_ref[e], preferred_element_type=jnp.float32)
        + b1_ref[e], 0.0)
    d = jnp.dot(h, w2_ref[e], preferred_element_type=jnp.float32) + b2_ref[e]
    acc = jnp.where(seg == e, d, acc)
  out_ref[...] = x[:, :O_PAD] + acc


@jax.jit
def _tc_mlp(feat, w1p, b1p, w2p, b2p):
  bm = 512
  grid = (N // bm,)
  return pl.pallas_call(
      _tc_mlp_body,
      grid=grid,
      in_specs=[
          pl.BlockSpec((bm, F_PAD), lambda i: (i, 0)),
          pl.BlockSpec((E, F_PAD, HID), lambda i: (0, 0, 0)),
          pl.BlockSpec((E, 1, HID), lambda i: (0, 0, 0)),
          pl.BlockSpec((E, HID, O_PAD), lambda i: (0, 0, 0)),
          pl.BlockSpec((E, 1, O_PAD), lambda i: (0, 0, 0)),
      ],
      out_specs=pl.BlockSpec((bm, O_PAD), lambda i: (i, 0)),
      out_shape=jax.ShapeDtypeStruct((N, O_PAD), jnp.float32),
  )(feat, w1p, b1p, w2p, b2p)


def kernel(idx_tensor, time_values, xyz, scaling, rotation, opacity, shs,
           embedding, seg_id_g, W1, b1, W2, b2):
  idx2d = idx_tensor.reshape(N // CH, CH)
  tab16 = jnp.concatenate(
      [xyz, scaling, rotation, opacity,
       seg_id_g.astype(jnp.float32).reshape(M, 1),
       jnp.zeros((M, 4), jnp.float32)], axis=-1)
  tab48 = shs.reshape(M, SH * 3)
  tab32 = embedding
  g16, g48, g32 = _sc_gather(idx2d, tab16, tab48, tab32)

  t = time_values.reshape(N, 1)
  feat = jnp.concatenate(
      [g16, g48, g32, t, jnp.zeros((N, F_PAD - 97), jnp.float32)], axis=-1)

  z = lambda r: jnp.zeros((E, r, HID), jnp.float32)
  w1p = jnp.concatenate(
      [W1[:, 0:11], z(5), W1[:, 11:59], W1[:, 59:91], W1[:, 91:92],
       z(F_PAD - 97)], axis=1)
  b1p = b1.reshape(E, 1, HID)
  zc = lambda c: jnp.zeros((E, HID, c), jnp.float32)
  w2p = jnp.concatenate([W2[:, :, 0:11], zc(5), W2[:, :, 11:59]], axis=2)
  zb = lambda c: jnp.zeros((E, c), jnp.float32)
  b2p = jnp.concatenate([b2[:, 0:11], zb(5), b2[:, 11:59]],
                        axis=1).reshape(E, 1, O_PAD)

  out = _tc_mlp(feat, w1p, b1p, w2p, b2p)

  means_o = out[:, 0:3]
  scales_o = out[:, 3:6]
  rot_o = out[:, 6:10]
  opa_o = out[:, 10:11]
  shs_o = out[:, 16:64].reshape(N, SH, 3)
  return (means_o, scales_o, rot_o, opa_o, shs_o)
